# baseline (speedup 1.0000x reference)
import jax
import jax.numpy as jnp
from jax import lax
from jax.experimental import pallas as pl
from jax.experimental.pallas import tpu as pltpu
from jax.experimental.pallas import tpu_sc as plsc

_CHUNK = 128
_L = 16


def _glove_sc(B, D, NC, NS):
    NW = NC * NS
    bpw = B // NW
    n_chunks = bpw // _CHUNK
    mesh = plsc.VectorSubcoreMesh(
        core_axis_name="c", subcore_axis_name="s",
        num_cores=NC, num_subcores=NS)

    def body(left_hbm, right_hbm, l_emb, l_bias, r_emb, r_bias, out_hbm,
             idx_l, idx_r, row_l, row_r, rows_l, rows_r, bl, br, outv, sem):
        wid = lax.axis_index("s") * NC + lax.axis_index("c")
        cbase = wid * n_chunks
        pltpu.sync_copy(left_hbm.at[pl.ds(cbase, n_chunks)], idx_l)
        pltpu.sync_copy(right_hbm.at[pl.ds(cbase, n_chunks)], idx_r)

        for j in range(n_chunks):
            def sbody(k, _):
                sl = pl.ds(k * _L, _L)
                row_l[j, sl] = lax.shift_right_logical(idx_l[j, sl], 4)
                row_r[j, sl] = lax.shift_right_logical(idx_r[j, sl], 4)
                return 0
            lax.fori_loop(0, _CHUNK // _L, sbody, 0)

        handles = []
        for j in range(n_chunks):
            sl = pl.ds(j * _CHUNK, _CHUNK)
            handles.append(pltpu.async_copy(l_emb.at[idx_l.at[j]], rows_l.at[sl], sem))
            handles.append(pltpu.async_copy(r_emb.at[idx_r.at[j]], rows_r.at[sl], sem))
            handles.append(pltpu.async_copy(l_bias.at[row_l.at[j]], bl.at[sl], sem))
            handles.append(pltpu.async_copy(r_bias.at[row_r.at[j]], br.at[sl], sem))
        for h in handles:
            h.wait()

        iota16 = lax.iota(jnp.int32, _L)
        for j in range(n_chunks):
            def gbody(k, _):
                base = j * _CHUNK + k * _L
                lane = base + iota16

                def dbody(dd, acc):
                    col = jnp.full((_L,), dd, jnp.int32)
                    lv = plsc.load_gather(rows_l, [lane, col])
                    rv = plsc.load_gather(rows_r, [lane, col])
                    return acc + lv * rv

                acc = lax.fori_loop(0, D, dbody, jnp.zeros((_L,), jnp.float32))
                sl = pl.ds(k * _L, _L)
                blv = plsc.load_gather(bl, [lane, jnp.bitwise_and(idx_l[j, sl], 15)])
                brv = plsc.load_gather(br, [lane, jnp.bitwise_and(idx_r[j, sl], 15)])
                outv[pl.ds(base, _L)] = acc + blv + brv
                return 0
            lax.fori_loop(0, _CHUNK // _L, gbody, 0)

        pltpu.sync_copy(outv, out_hbm.at[pl.ds(wid * bpw, bpw)])

    return pl.kernel(
        body,
        out_type=jax.ShapeDtypeStruct((B,), jnp.float32),
        mesh=mesh,
        compiler_params=pltpu.CompilerParams(
            needs_layout_passes=False, use_tc_tiling_on_sc=False),
        scratch_types=[
            pltpu.VMEM((n_chunks, _CHUNK), jnp.int32),
            pltpu.VMEM((n_chunks, _CHUNK), jnp.int32),
            pltpu.VMEM((n_chunks, _CHUNK), jnp.int32),
            pltpu.VMEM((n_chunks, _CHUNK), jnp.int32),
            pltpu.VMEM((bpw, D), jnp.float32),
            pltpu.VMEM((bpw, D), jnp.float32),
            pltpu.VMEM((bpw, _L), jnp.float32),
            pltpu.VMEM((bpw, _L), jnp.float32),
            pltpu.VMEM((bpw,), jnp.float32),
            pltpu.SemaphoreType.DMA,
        ],
    )


def kernel(left, right, l_emb, l_bias, r_emb, r_bias):
    (B,) = left.shape
    V, D = l_emb.shape
    info = plsc.get_sparse_core_info()
    NC, NS = info.num_cores, info.num_subcores
    left2d = left.astype(jnp.int32).reshape(B // _CHUNK, _CHUNK)
    right2d = right.astype(jnp.int32).reshape(B // _CHUNK, _CHUNK)
    lb2d = l_bias.reshape(V // _L, _L)
    rb2d = r_bias.reshape(V // _L, _L)
    fn = _glove_sc(B, D, NC, NS)
    return fn(left2d, right2d, l_emb, lb2d, r_emb, rb2d)
